# Initial kernel scaffold; baseline (speedup 1.0000x reference)
import jax
import jax.numpy as jnp
from jax.experimental import pallas as pl


def kernel(input_q, input_kv, Wq, Wk):
    raise NotImplementedError("write your pallas kernel here")


# speedup vs baseline: 1.6677x; 1.6677x over previous
import functools
import math

import jax
import jax.numpy as jnp
from jax.experimental import pallas as pl
from jax.experimental.pallas import tpu as pltpu

SEQ = 16
B = 8
D_IN = 512
D_QKV = 512
N = 512
R = SEQ * B
G = 8


def _body(xq_ref, wq_ref, wk_ref, kv_ref, out_ref):
    q = jax.lax.dot_general(
        xq_ref[...], wq_ref[...],
        dimension_numbers=(((1,), (1,)), ((), ())),
        preferred_element_type=jnp.float32,
    )
    qt = jax.lax.dot_general(
        q, wk_ref[...],
        dimension_numbers=(((1,), (0,)), ((), ())),
        preferred_element_type=jnp.float32,
    ) * (1.0 / math.sqrt(D_QKV))
    out_ref[...] = jnp.sum(kv_ref[...] * qt[:, None, :], axis=-1)


@jax.jit
def _run(xq, kv, Wq, Wk):
    return pl.pallas_call(
        _body,
        grid=(R // G,),
        in_specs=[
            pl.BlockSpec((G, D_IN), lambda i: (i, 0)),
            pl.BlockSpec((D_QKV, D_IN), lambda i: (0, 0)),
            pl.BlockSpec((D_QKV, D_IN), lambda i: (0, 0)),
            pl.BlockSpec((G, N, D_IN), lambda i: (i, 0, 0)),
        ],
        out_specs=pl.BlockSpec((G, N), lambda i: (i, 0)),
        out_shape=jax.ShapeDtypeStruct((R, N), jnp.float32),
    )(xq, Wq, Wk, kv)


def kernel(input_q, input_kv, Wq, Wk):
    xq = input_q.reshape(R, D_IN)
    kv = input_kv.reshape(R, N, D_IN)
    qk = _run(xq, kv, Wq, Wk)
    return qk.reshape(SEQ, B, N)
